# baseline (speedup 1.0000x reference)
import functools

import jax
import jax.numpy as jnp
from jax import lax
from jax.experimental import pallas as pl
from jax.experimental.pallas import tpu as pltpu
from jax.experimental.pallas import tpu_sc as plsc

N = 10000
D = 128
H = 128
E = 320000

NC = 2
NS = 16
C = 128
ROWS0 = 632
ROWS_LAST = N - (NS - 1) * ROWS0
E_SC = E // NC
CH_SC = E_SC // C
CH_PER_TILE = -(-CH_SC // NS)

_MESH = plsc.VectorSubcoreMesh(
    core_axis_name="c", subcore_axis_name="s", num_cores=NC, num_subcores=NS
)


def _fill_rows(ref, n_rows, n_cols, value):
    vec = jnp.full((16,), value, dtype=ref.dtype)

    def body(i, carry):
        for j in range(n_cols // 16):
            ref[i, pl.ds(j * 16, 16)] = vec
        return carry

    lax.fori_loop(0, n_rows, body, 0)


def _zero_owned_rows(sh, zbuf, s):
    @pl.when(s < NS - 1)
    def _():
        base = s * ROWS0
        for j in range(4):
            pltpu.sync_copy(zbuf.at[pl.ds(0, 128)], sh.at[pl.ds(base + j * 128, 128)])
        pltpu.sync_copy(zbuf.at[pl.ds(0, ROWS0 - 512)], sh.at[pl.ds(base + 512, ROWS0 - 512)])

    @pl.when(s == NS - 1)
    def _():
        base = (NS - 1) * ROWS0
        for j in range(4):
            pltpu.sync_copy(zbuf.at[pl.ds(0, 128)], sh.at[pl.ds(base + j * 128, 128)])
        pltpu.sync_copy(zbuf.at[pl.ds(0, ROWS_LAST - 512)], sh.at[pl.ds(base + 512, ROWS_LAST - 512)])


def _write_owned_rows(sh, out, c, s):
    @pl.when(s < NS - 1)
    def _():
        pltpu.sync_copy(
            sh.at[pl.ds(s * ROWS0, ROWS0)],
            out.at[pl.ds(c * N + s * ROWS0, ROWS0)],
        )

    @pl.when(s == NS - 1)
    def _():
        pltpu.sync_copy(
            sh.at[pl.ds((NS - 1) * ROWS0, ROWS_LAST)],
            out.at[pl.ds(c * N + (NS - 1) * ROWS0, ROWS_LAST)],
        )


@functools.partial(
    pl.kernel,
    out_type=jax.ShapeDtypeStruct((2 * N, H), jnp.float32),
    mesh=_MESH,
    scratch_types=[
        pltpu.VMEM_SHARED((N, H), jnp.float32),
        pltpu.VMEM((C, H), jnp.float32),
        pltpu.VMEM((C,), jnp.int32),
        pltpu.VMEM((C,), jnp.int32),
        pltpu.SemaphoreType.DMA,
        pltpu.SemaphoreType.DMA,
    ],
)
def _hist(dst_hbm, out_hbm, deg_sh, vbuf, idx0, idx1, sem0, sem1):
    c = lax.axis_index("c")
    s = lax.axis_index("s")

    _fill_rows(vbuf, C, H, 0.0)
    _zero_owned_rows(deg_sh, vbuf, s)
    _fill_rows(vbuf, C, H, 1.0)
    plsc.subcore_barrier()

    idx = (idx0, idx1)
    sem = (sem0, sem1)

    def start(i, b):
        cid = s + i * NS

        @pl.when(cid < CH_SC)
        def _():
            base = c * E_SC + cid * C
            pltpu.async_copy(dst_hbm.at[pl.ds(base, C)], idx[b], sem[b])

    def finish(i, b):
        cid = s + i * NS

        @pl.when(cid < CH_SC)
        def _():
            base = c * E_SC + cid * C
            pltpu.make_async_copy(dst_hbm.at[pl.ds(base, C)], idx[b], sem[b]).wait()
            pltpu.sync_copy(vbuf, deg_sh.at[idx[b]], add=True)

    start(0, 0)

    def body(p, carry):
        i0 = 2 * p
        start(i0 + 1, 1)
        finish(i0, 0)
        start(i0 + 2, 0)
        finish(i0 + 1, 1)
        return carry

    lax.fori_loop(0, (CH_PER_TILE + 1) // 2, body, 0)
    plsc.subcore_barrier()

    _write_owned_rows(deg_sh, out_hbm, c, s)


@functools.partial(
    pl.kernel,
    out_type=jax.ShapeDtypeStruct((2 * N, H), jnp.float32),
    mesh=_MESH,
    scratch_types=[
        pltpu.VMEM_SHARED((N, H), jnp.float32),
        [pltpu.VMEM((C, H), jnp.float32)] * 3,
        [pltpu.VMEM((C,), jnp.int32)] * 3,
        [pltpu.VMEM((C,), jnp.int32)] * 3,
        [pltpu.SemaphoreType.DMA] * 3,
        [pltpu.SemaphoreType.DMA] * 3,
    ],
)
def _agg(g_hbm, src_hbm, dst_hbm, out_hbm, agg_sh, msg, sidx, didx, isem, gsem):
    c = lax.axis_index("c")
    s = lax.axis_index("s")
    NB = 3

    _fill_rows(msg[0], C, H, 0.0)
    _zero_owned_rows(agg_sh, msg[0], s)
    plsc.subcore_barrier()

    def issue_idx(i, b):
        cid = s + i * NS

        @pl.when(cid < CH_SC)
        def _():
            base = c * E_SC + cid * C
            pltpu.async_copy(src_hbm.at[pl.ds(base, C)], sidx[b], isem[b])
            pltpu.async_copy(dst_hbm.at[pl.ds(base, C)], didx[b], isem[b])

    def issue_gather(i, b):
        cid = s + i * NS

        @pl.when(cid < CH_SC)
        def _():
            base = c * E_SC + cid * C
            pltpu.make_async_copy(src_hbm.at[pl.ds(base, C)], sidx[b], isem[b]).wait()
            pltpu.make_async_copy(dst_hbm.at[pl.ds(base, C)], didx[b], isem[b]).wait()
            pltpu.async_copy(g_hbm.at[sidx[b]], msg[b], gsem[b])

    def finish(i, b):
        cid = s + i * NS

        @pl.when(cid < CH_SC)
        def _():
            pltpu.make_async_copy(g_hbm.at[sidx[b]], msg[b], gsem[b]).wait()
            pltpu.sync_copy(msg[b], agg_sh.at[didx[b]], add=True)

    issue_idx(0, 0)
    issue_idx(1, 1)
    issue_gather(0, 0)

    def body(p, carry):
        i0 = NB * p
        for r in range(NB):
            issue_idx(i0 + r + 2, (r + 2) % NB)
            issue_gather(i0 + r + 1, (r + 1) % NB)
            finish(i0 + r, r)
        return carry

    lax.fori_loop(0, -(-CH_PER_TILE // NB), body, 0)
    plsc.subcore_barrier()

    _write_owned_rows(agg_sh, out_hbm, c, s)


_RB = 2000
_GRID = N // _RB


def _mm1_body(x_ref, w_ref, h_ref):
    h_ref[...] = jnp.dot(x_ref[...], w_ref[...], preferred_element_type=jnp.float32)


def _mm1(x, w1):
    return pl.pallas_call(
        _mm1_body,
        grid=(_GRID,),
        in_specs=[
            pl.BlockSpec((_RB, D), lambda i: (i, 0)),
            pl.BlockSpec((D, H), lambda i: (0, 0)),
        ],
        out_specs=pl.BlockSpec((_RB, H), lambda i: (i, 0)),
        out_shape=jax.ShapeDtypeStruct((N, H), jnp.float32),
    )(x, w1)


def _scale_body(h_ref, degp_ref, g_ref, dinv_ref):
    deg = degp_ref[0] + degp_ref[1]
    dinv = lax.rsqrt(deg[:, 0:1] + 1.0)
    g_ref[...] = h_ref[...] * dinv
    dinv_ref[...] = dinv


def _scale(h, degp):
    return pl.pallas_call(
        _scale_body,
        grid=(_GRID,),
        in_specs=[
            pl.BlockSpec((_RB, H), lambda i: (i, 0)),
            pl.BlockSpec((2, _RB, H), lambda i: (0, i, 0)),
        ],
        out_specs=[
            pl.BlockSpec((_RB, H), lambda i: (i, 0)),
            pl.BlockSpec((_RB, 1), lambda i: (i, 0)),
        ],
        out_shape=[
            jax.ShapeDtypeStruct((N, H), jnp.float32),
            jax.ShapeDtypeStruct((N, 1), jnp.float32),
        ],
    )(h, degp)


def _mid_body(aggp_ref, g1_ref, dinv_ref, b1_ref, w2_ref, g2_ref):
    dinv = dinv_ref[...]
    a = aggp_ref[0] + aggp_ref[1] + g1_ref[...]
    x2 = jnp.maximum(a * dinv + b1_ref[...], 0.0)
    h2 = jnp.dot(x2, w2_ref[...], preferred_element_type=jnp.float32)
    g2_ref[...] = h2 * dinv


def _mid(aggp, g1, dinv, b1, w2):
    return pl.pallas_call(
        _mid_body,
        grid=(_GRID,),
        in_specs=[
            pl.BlockSpec((2, _RB, H), lambda i: (0, i, 0)),
            pl.BlockSpec((_RB, H), lambda i: (i, 0)),
            pl.BlockSpec((_RB, 1), lambda i: (i, 0)),
            pl.BlockSpec((1, H), lambda i: (0, 0)),
            pl.BlockSpec((H, H), lambda i: (0, 0)),
        ],
        out_specs=pl.BlockSpec((_RB, H), lambda i: (i, 0)),
        out_shape=jax.ShapeDtypeStruct((N, H), jnp.float32),
    )(aggp, g1, dinv, b1, w2)


def _out_body(aggp_ref, g2_ref, dinv_ref, b2_ref, out_ref):
    a = aggp_ref[0] + aggp_ref[1] + g2_ref[...]
    out_ref[...] = a * dinv_ref[...] + b2_ref[...]


def _final(aggp, g2, dinv, b2):
    return pl.pallas_call(
        _out_body,
        grid=(_GRID,),
        in_specs=[
            pl.BlockSpec((2, _RB, H), lambda i: (0, i, 0)),
            pl.BlockSpec((_RB, H), lambda i: (i, 0)),
            pl.BlockSpec((_RB, 1), lambda i: (i, 0)),
            pl.BlockSpec((1, H), lambda i: (0, 0)),
        ],
        out_specs=pl.BlockSpec((_RB, H), lambda i: (i, 0)),
        out_shape=jax.ShapeDtypeStruct((N, H), jnp.float32),
    )(aggp, g2, dinv, b2)


def kernel(x, edge_index, W1, b1, W2, b2):
    src = edge_index[0]
    dst = edge_index[1]

    h1 = _mm1(x, W1)
    degp = _hist(dst).reshape(2, N, H)
    g1, dinv = _scale(h1, degp)
    agg1 = _agg(g1, src, dst).reshape(2, N, H)
    g2 = _mid(agg1, g1, dinv, b1.reshape(1, H), W2)
    agg2 = _agg(g2, src, dst).reshape(2, N, H)
    return _final(agg2, g2, dinv, b2.reshape(1, H))


# speedup vs baseline: 1.0017x; 1.0017x over previous
import functools

import jax
import jax.numpy as jnp
from jax import lax
from jax.experimental import pallas as pl
from jax.experimental.pallas import tpu as pltpu
from jax.experimental.pallas import tpu_sc as plsc

N = 10000
D = 128
H = 128
E = 320000

NC = 2
NS = 16
C = 128
ROWS0 = 632
ROWS_LAST = N - (NS - 1) * ROWS0
E_SC = E // NC
CH_SC = E_SC // C
CH_PER_TILE = -(-CH_SC // NS)

_MESH = plsc.VectorSubcoreMesh(
    core_axis_name="c", subcore_axis_name="s", num_cores=NC, num_subcores=NS
)


def _fill_rows(ref, n_rows, n_cols, value):
    vec = jnp.full((16,), value, dtype=ref.dtype)

    def body(i, carry):
        for j in range(n_cols // 16):
            ref[i, pl.ds(j * 16, 16)] = vec
        return carry

    lax.fori_loop(0, n_rows, body, 0)


def _zero_owned_rows(sh, zbuf, s):
    @pl.when(s < NS - 1)
    def _():
        base = s * ROWS0
        for j in range(4):
            pltpu.sync_copy(zbuf.at[pl.ds(0, 128)], sh.at[pl.ds(base + j * 128, 128)])
        pltpu.sync_copy(zbuf.at[pl.ds(0, ROWS0 - 512)], sh.at[pl.ds(base + 512, ROWS0 - 512)])

    @pl.when(s == NS - 1)
    def _():
        base = (NS - 1) * ROWS0
        for j in range(4):
            pltpu.sync_copy(zbuf.at[pl.ds(0, 128)], sh.at[pl.ds(base + j * 128, 128)])
        pltpu.sync_copy(zbuf.at[pl.ds(0, ROWS_LAST - 512)], sh.at[pl.ds(base + 512, ROWS_LAST - 512)])


def _write_owned_rows(sh, out, c, s):
    @pl.when(s < NS - 1)
    def _():
        pltpu.sync_copy(
            sh.at[pl.ds(s * ROWS0, ROWS0)],
            out.at[pl.ds(c * N + s * ROWS0, ROWS0)],
        )

    @pl.when(s == NS - 1)
    def _():
        pltpu.sync_copy(
            sh.at[pl.ds((NS - 1) * ROWS0, ROWS_LAST)],
            out.at[pl.ds(c * N + (NS - 1) * ROWS0, ROWS_LAST)],
        )


@functools.partial(
    pl.kernel,
    out_type=jax.ShapeDtypeStruct((2 * N, H), jnp.float32),
    mesh=_MESH,
    scratch_types=[
        pltpu.VMEM_SHARED((N, H), jnp.float32),
        pltpu.VMEM((C, H), jnp.float32),
        pltpu.VMEM((C,), jnp.int32),
        pltpu.VMEM((C,), jnp.int32),
        pltpu.SemaphoreType.DMA,
        pltpu.SemaphoreType.DMA,
    ],
)
def _hist(dst_hbm, out_hbm, deg_sh, vbuf, idx0, idx1, sem0, sem1):
    c = lax.axis_index("c")
    s = lax.axis_index("s")

    _fill_rows(vbuf, C, H, 0.0)
    _zero_owned_rows(deg_sh, vbuf, s)
    _fill_rows(vbuf, C, H, 1.0)
    plsc.subcore_barrier()

    idx = (idx0, idx1)
    sem = (sem0, sem1)

    def start(i, b):
        cid = s + i * NS

        @pl.when(cid < CH_SC)
        def _():
            base = c * E_SC + cid * C
            pltpu.async_copy(dst_hbm.at[pl.ds(base, C)], idx[b], sem[b])

    def finish(i, b):
        cid = s + i * NS

        @pl.when(cid < CH_SC)
        def _():
            base = c * E_SC + cid * C
            pltpu.make_async_copy(dst_hbm.at[pl.ds(base, C)], idx[b], sem[b]).wait()
            pltpu.sync_copy(vbuf, deg_sh.at[idx[b]], add=True)

    start(0, 0)

    def body(p, carry):
        i0 = 2 * p
        start(i0 + 1, 1)
        finish(i0, 0)
        start(i0 + 2, 0)
        finish(i0 + 1, 1)
        return carry

    lax.fori_loop(0, (CH_PER_TILE + 1) // 2, body, 0)
    plsc.subcore_barrier()

    _write_owned_rows(deg_sh, out_hbm, c, s)


@functools.partial(
    pl.kernel,
    out_type=jax.ShapeDtypeStruct((2 * N, H), jnp.float32),
    mesh=_MESH,
    scratch_types=[
        pltpu.VMEM_SHARED((N, H), jnp.float32),
        [pltpu.VMEM((C, H), jnp.float32)] * 3,
        [pltpu.VMEM((C,), jnp.int32)] * 3,
        [pltpu.VMEM((C,), jnp.int32)] * 3,
        [pltpu.SemaphoreType.DMA] * 3,
        [pltpu.SemaphoreType.DMA] * 3,
    ],
)
def _agg(g_hbm, src_hbm, dst_hbm, out_hbm, agg_sh, msg, sidx, didx, isem, gsem):
    c = lax.axis_index("c")
    s = lax.axis_index("s")
    NB = 3

    _fill_rows(msg[0], C, H, 0.0)
    _zero_owned_rows(agg_sh, msg[0], s)
    plsc.subcore_barrier()

    def issue_idx(i, b):
        cid = s + i * NS

        @pl.when(cid < CH_SC)
        def _():
            base = c * E_SC + cid * C
            pltpu.async_copy(src_hbm.at[pl.ds(base, C)], sidx[b], isem[b])
            pltpu.async_copy(dst_hbm.at[pl.ds(base, C)], didx[b], isem[b])

    def issue_gather(i, b):
        cid = s + i * NS

        @pl.when(cid < CH_SC)
        def _():
            base = c * E_SC + cid * C
            pltpu.make_async_copy(src_hbm.at[pl.ds(base, C)], sidx[b], isem[b]).wait()
            pltpu.make_async_copy(dst_hbm.at[pl.ds(base, C)], didx[b], isem[b]).wait()
            pltpu.async_copy(g_hbm.at[sidx[b]], msg[b], gsem[b])

    def finish(i, b):
        cid = s + i * NS

        @pl.when(cid < CH_SC)
        def _():
            pltpu.make_async_copy(g_hbm.at[sidx[b]], msg[b], gsem[b]).wait()
            pltpu.sync_copy(msg[b], agg_sh.at[didx[b]], add=True)

    issue_idx(0, 0)
    issue_idx(1, 1)
    issue_gather(0, 0)

    def body(p, carry):
        i0 = NB * p
        for r in range(NB):
            issue_idx(i0 + r + 2, (r + 2) % NB)
            issue_gather(i0 + r + 1, (r + 1) % NB)
            finish(i0 + r, r)
        return carry

    lax.fori_loop(0, -(-CH_PER_TILE // NB), body, 0)
    plsc.subcore_barrier()

    _write_owned_rows(agg_sh, out_hbm, c, s)


_RB = 2000
_GRID = N // _RB


def _mm_scale_body(x_ref, w_ref, degp_ref, g_ref, dinv_ref):
    deg = degp_ref[0] + degp_ref[1]
    dinv = lax.rsqrt(deg[:, 0:1] + 1.0)
    h = jnp.dot(x_ref[...], w_ref[...], preferred_element_type=jnp.float32)
    g_ref[...] = h * dinv
    dinv_ref[...] = dinv


def _mm_scale(x, w1, degp):
    return pl.pallas_call(
        _mm_scale_body,
        grid=(_GRID,),
        in_specs=[
            pl.BlockSpec((_RB, D), lambda i: (i, 0)),
            pl.BlockSpec((D, H), lambda i: (0, 0)),
            pl.BlockSpec((2, _RB, H), lambda i: (0, i, 0)),
        ],
        out_specs=[
            pl.BlockSpec((_RB, H), lambda i: (i, 0)),
            pl.BlockSpec((_RB, 1), lambda i: (i, 0)),
        ],
        out_shape=[
            jax.ShapeDtypeStruct((N, H), jnp.float32),
            jax.ShapeDtypeStruct((N, 1), jnp.float32),
        ],
    )(x, w1, degp)


def _mid_body(aggp_ref, g1_ref, dinv_ref, b1_ref, w2_ref, g2_ref):
    dinv = dinv_ref[...]
    a = aggp_ref[0] + aggp_ref[1] + g1_ref[...]
    x2 = jnp.maximum(a * dinv + b1_ref[...], 0.0)
    h2 = jnp.dot(x2, w2_ref[...], preferred_element_type=jnp.float32)
    g2_ref[...] = h2 * dinv


def _mid(aggp, g1, dinv, b1, w2):
    return pl.pallas_call(
        _mid_body,
        grid=(_GRID,),
        in_specs=[
            pl.BlockSpec((2, _RB, H), lambda i: (0, i, 0)),
            pl.BlockSpec((_RB, H), lambda i: (i, 0)),
            pl.BlockSpec((_RB, 1), lambda i: (i, 0)),
            pl.BlockSpec((1, H), lambda i: (0, 0)),
            pl.BlockSpec((H, H), lambda i: (0, 0)),
        ],
        out_specs=pl.BlockSpec((_RB, H), lambda i: (i, 0)),
        out_shape=jax.ShapeDtypeStruct((N, H), jnp.float32),
    )(aggp, g1, dinv, b1, w2)


def _out_body(aggp_ref, g2_ref, dinv_ref, b2_ref, out_ref):
    a = aggp_ref[0] + aggp_ref[1] + g2_ref[...]
    out_ref[...] = a * dinv_ref[...] + b2_ref[...]


def _final(aggp, g2, dinv, b2):
    return pl.pallas_call(
        _out_body,
        grid=(_GRID,),
        in_specs=[
            pl.BlockSpec((2, _RB, H), lambda i: (0, i, 0)),
            pl.BlockSpec((_RB, H), lambda i: (i, 0)),
            pl.BlockSpec((_RB, 1), lambda i: (i, 0)),
            pl.BlockSpec((1, H), lambda i: (0, 0)),
        ],
        out_specs=pl.BlockSpec((_RB, H), lambda i: (i, 0)),
        out_shape=jax.ShapeDtypeStruct((N, H), jnp.float32),
    )(aggp, g2, dinv, b2)


def kernel(x, edge_index, W1, b1, W2, b2):
    src = edge_index[0]
    dst = edge_index[1]

    degp = _hist(dst).reshape(2, N, H)
    g1, dinv = _mm_scale(x, W1, degp)
    agg1 = _agg(g1, src, dst).reshape(2, N, H)
    g2 = _mid(agg1, g1, dinv, b1.reshape(1, H), W2)
    agg2 = _agg(g2, src, dst).reshape(2, N, H)
    return _final(agg2, g2, dinv, b2.reshape(1, H))


# speedup vs baseline: 1.0770x; 1.0751x over previous
import functools

import jax
import jax.numpy as jnp
from jax import lax
from jax.experimental import pallas as pl
from jax.experimental.pallas import tpu as pltpu
from jax.experimental.pallas import tpu_sc as plsc

N = 10000
D = 128
H = 128
E = 320000

NC = 2
NS = 16
C = 128
ROWS0 = 632
ROWS_LAST = N - (NS - 1) * ROWS0
E_SC = E // NC
CH_SC = E_SC // C
CH_PER_TILE = -(-CH_SC // NS)

_MESH = plsc.VectorSubcoreMesh(
    core_axis_name="c", subcore_axis_name="s", num_cores=NC, num_subcores=NS
)


def _fill_rows(ref, n_rows, n_cols, value):
    vec = jnp.full((16,), value, dtype=ref.dtype)

    def body(i, carry):
        for j in range(n_cols // 16):
            ref[i, pl.ds(j * 16, 16)] = vec
        return carry

    lax.fori_loop(0, n_rows, body, 0)


def _zero_owned_rows(sh, zbuf, s):
    @pl.when(s < NS - 1)
    def _():
        base = s * ROWS0
        for j in range(4):
            pltpu.sync_copy(zbuf.at[pl.ds(0, 128)], sh.at[pl.ds(base + j * 128, 128)])
        pltpu.sync_copy(zbuf.at[pl.ds(0, ROWS0 - 512)], sh.at[pl.ds(base + 512, ROWS0 - 512)])

    @pl.when(s == NS - 1)
    def _():
        base = (NS - 1) * ROWS0
        for j in range(4):
            pltpu.sync_copy(zbuf.at[pl.ds(0, 128)], sh.at[pl.ds(base + j * 128, 128)])
        pltpu.sync_copy(zbuf.at[pl.ds(0, ROWS_LAST - 512)], sh.at[pl.ds(base + 512, ROWS_LAST - 512)])


def _write_owned_rows(sh, out, c, s):
    @pl.when(s < NS - 1)
    def _():
        pltpu.sync_copy(
            sh.at[pl.ds(s * ROWS0, ROWS0)],
            out.at[pl.ds(c * N + s * ROWS0, ROWS0)],
        )

    @pl.when(s == NS - 1)
    def _():
        pltpu.sync_copy(
            sh.at[pl.ds((NS - 1) * ROWS0, ROWS_LAST)],
            out.at[pl.ds(c * N + (NS - 1) * ROWS0, ROWS_LAST)],
        )


@functools.partial(
    pl.kernel,
    out_type=jax.ShapeDtypeStruct((2 * N, H), jnp.float32),
    mesh=_MESH,
    scratch_types=[
        pltpu.VMEM_SHARED((N, H), jnp.float32),
        pltpu.VMEM((C, H), jnp.float32),
        pltpu.VMEM((C,), jnp.int32),
        pltpu.VMEM((C,), jnp.int32),
        pltpu.SemaphoreType.DMA,
        pltpu.SemaphoreType.DMA,
    ],
)
def _hist(dst_hbm, out_hbm, deg_sh, vbuf, idx0, idx1, sem0, sem1):
    c = lax.axis_index("c")
    s = lax.axis_index("s")

    _fill_rows(vbuf, C, H, 0.0)
    _zero_owned_rows(deg_sh, vbuf, s)
    _fill_rows(vbuf, C, H, 1.0)
    plsc.subcore_barrier()

    idx = (idx0, idx1)
    sem = (sem0, sem1)

    def start(i, b):
        cid = s + i * NS

        @pl.when(cid < CH_SC)
        def _():
            base = c * E_SC + cid * C
            pltpu.async_copy(dst_hbm.at[pl.ds(base, C)], idx[b], sem[b])

    def finish(i, b):
        cid = s + i * NS

        @pl.when(cid < CH_SC)
        def _():
            base = c * E_SC + cid * C
            pltpu.make_async_copy(dst_hbm.at[pl.ds(base, C)], idx[b], sem[b]).wait()
            pltpu.sync_copy(vbuf, deg_sh.at[idx[b]], add=True)

    start(0, 0)

    def body(p, carry):
        i0 = 2 * p
        start(i0 + 1, 1)
        finish(i0, 0)
        start(i0 + 2, 0)
        finish(i0 + 1, 1)
        return carry

    lax.fori_loop(0, (CH_PER_TILE + 1) // 2, body, 0)
    plsc.subcore_barrier()

    _write_owned_rows(deg_sh, out_hbm, c, s)


@functools.partial(
    pl.kernel,
    out_type=jax.ShapeDtypeStruct((2 * N, H), jnp.float32),
    mesh=_MESH,
    scratch_types=[
        pltpu.VMEM_SHARED((N, H), jnp.float32),
        [pltpu.VMEM((C, H), jnp.float32)] * 3,
        [pltpu.VMEM((C,), jnp.int32)] * 4,
        [pltpu.VMEM((C,), jnp.int32)] * 4,
        [pltpu.SemaphoreType.DMA] * 4,
        [pltpu.SemaphoreType.DMA] * 3,
    ],
)
def _agg(g_hbm, src_hbm, dst_hbm, out_hbm, agg_sh, msg, sidx, didx, isem, gsem):
    c = lax.axis_index("c")
    s = lax.axis_index("s")
    NB = 3

    _fill_rows(msg[0], C, H, 0.0)
    _zero_owned_rows(agg_sh, msg[0], s)
    plsc.subcore_barrier()

    def issue_idx(i, b):
        cid = s + i * NS

        @pl.when(cid < CH_SC)
        def _():
            base = c * E_SC + cid * C
            pltpu.async_copy(src_hbm.at[pl.ds(base, C)], sidx[b], isem[b])
            pltpu.async_copy(dst_hbm.at[pl.ds(base, C)], didx[b], isem[b])

    def issue_gather(i, b, m):
        cid = s + i * NS

        @pl.when(cid < CH_SC)
        def _():
            base = c * E_SC + cid * C
            pltpu.make_async_copy(src_hbm.at[pl.ds(base, C)], sidx[b], isem[b]).wait()
            pltpu.make_async_copy(dst_hbm.at[pl.ds(base, C)], didx[b], isem[b]).wait()
            pltpu.async_copy(g_hbm.at[sidx[b]], msg[m], gsem[m])

    def finish(i, b, m):
        cid = s + i * NS

        @pl.when(cid < CH_SC)
        def _():
            pltpu.make_async_copy(g_hbm.at[sidx[b]], msg[m], gsem[m]).wait()
            pltpu.sync_copy(msg[m], agg_sh.at[didx[b]], add=True)

    issue_idx(0, 0)
    issue_idx(1, 1)
    issue_idx(2, 2)
    issue_gather(0, 0, 0)
    issue_gather(1, 1, 1)

    STEP = 12

    def body(p, carry):
        i0 = STEP * p
        for r in range(STEP):
            issue_idx(i0 + r + 3, (r + 3) % 4)
            issue_gather(i0 + r + 2, (r + 2) % 4, (r + 2) % NB)
            finish(i0 + r, r % 4, r % NB)
        return carry

    lax.fori_loop(0, -(-CH_PER_TILE // STEP), body, 0)
    plsc.subcore_barrier()

    _write_owned_rows(agg_sh, out_hbm, c, s)


_RB = 2000
_GRID = N // _RB


def _mm_scale_body(x_ref, w_ref, degp_ref, g_ref, dinv_ref):
    deg = degp_ref[0] + degp_ref[1]
    dinv = lax.rsqrt(deg[:, 0:1] + 1.0)
    h = jnp.dot(x_ref[...], w_ref[...], preferred_element_type=jnp.float32)
    g_ref[...] = h * dinv
    dinv_ref[...] = dinv


def _mm_scale(x, w1, degp):
    return pl.pallas_call(
        _mm_scale_body,
        grid=(_GRID,),
        in_specs=[
            pl.BlockSpec((_RB, D), lambda i: (i, 0)),
            pl.BlockSpec((D, H), lambda i: (0, 0)),
            pl.BlockSpec((2, _RB, H), lambda i: (0, i, 0)),
        ],
        out_specs=[
            pl.BlockSpec((_RB, H), lambda i: (i, 0)),
            pl.BlockSpec((_RB, 1), lambda i: (i, 0)),
        ],
        out_shape=[
            jax.ShapeDtypeStruct((N, H), jnp.float32),
            jax.ShapeDtypeStruct((N, 1), jnp.float32),
        ],
    )(x, w1, degp)


def _mid_body(aggp_ref, g1_ref, dinv_ref, b1_ref, w2_ref, g2_ref):
    dinv = dinv_ref[...]
    a = aggp_ref[0] + aggp_ref[1] + g1_ref[...]
    x2 = jnp.maximum(a * dinv + b1_ref[...], 0.0)
    h2 = jnp.dot(x2, w2_ref[...], preferred_element_type=jnp.float32)
    g2_ref[...] = h2 * dinv


def _mid(aggp, g1, dinv, b1, w2):
    return pl.pallas_call(
        _mid_body,
        grid=(_GRID,),
        in_specs=[
            pl.BlockSpec((2, _RB, H), lambda i: (0, i, 0)),
            pl.BlockSpec((_RB, H), lambda i: (i, 0)),
            pl.BlockSpec((_RB, 1), lambda i: (i, 0)),
            pl.BlockSpec((1, H), lambda i: (0, 0)),
            pl.BlockSpec((H, H), lambda i: (0, 0)),
        ],
        out_specs=pl.BlockSpec((_RB, H), lambda i: (i, 0)),
        out_shape=jax.ShapeDtypeStruct((N, H), jnp.float32),
    )(aggp, g1, dinv, b1, w2)


def _out_body(aggp_ref, g2_ref, dinv_ref, b2_ref, out_ref):
    a = aggp_ref[0] + aggp_ref[1] + g2_ref[...]
    out_ref[...] = a * dinv_ref[...] + b2_ref[...]


def _final(aggp, g2, dinv, b2):
    return pl.pallas_call(
        _out_body,
        grid=(_GRID,),
        in_specs=[
            pl.BlockSpec((2, _RB, H), lambda i: (0, i, 0)),
            pl.BlockSpec((_RB, H), lambda i: (i, 0)),
            pl.BlockSpec((_RB, 1), lambda i: (i, 0)),
            pl.BlockSpec((1, H), lambda i: (0, 0)),
        ],
        out_specs=pl.BlockSpec((_RB, H), lambda i: (i, 0)),
        out_shape=jax.ShapeDtypeStruct((N, H), jnp.float32),
    )(aggp, g2, dinv, b2)


def kernel(x, edge_index, W1, b1, W2, b2):
    src = edge_index[0]
    dst = edge_index[1]

    degp = _hist(dst).reshape(2, N, H)
    g1, dinv = _mm_scale(x, W1, degp)
    agg1 = _agg(g1, src, dst).reshape(2, N, H)
    g2 = _mid(agg1, g1, dinv, b1.reshape(1, H), W2)
    agg2 = _agg(g2, src, dst).reshape(2, N, H)
    return _final(agg2, g2, dinv, b2.reshape(1, H))
